# baseline (speedup 1.0000x reference)
import functools

import jax
import jax.numpy as jnp
import numpy as np
from jax import lax
from jax.experimental import pallas as pl
from jax.experimental.pallas import tpu as pltpu
from jax.experimental.pallas import tpu_sc as plsc

N = 8192
M = 2048
K = 16
CIN = 256
COUT = 512

_FR, _FC = 8, N // 8


def _argmax_pair_tree(dd, ll):
    def fold(da, la, db, lb):
        better = (da > db) | ((da == db) & (la < lb))
        return jnp.where(better, da, db), jnp.where(better, la, lb)

    w = dd.shape[1]
    while w > 1:
        h = w // 2
        dd, ll = fold(dd[:, :h], ll[:, :h], dd[:, h:w], ll[:, h:w])
        w = h
    s = dd.shape[0]
    while s > 1:
        h = s // 2
        dd, ll = fold(dd[:h], ll[:h], dd[h:s], ll[h:s])
        s = h
    return ll[0, 0]


def _fps_body(px_ref, py_ref, pz_ref, pxs_ref, pys_ref, pzs_ref,
              sel_ref, qx_ref, qy_ref, qz_ref):
    px = px_ref[...]
    py = py_ref[...]
    pz = pz_ref[...]
    riota = lax.broadcasted_iota(jnp.int32, (_FR, _FC), 0)
    lane1 = lax.broadcasted_iota(jnp.int32, (1, _FC), 1)

    sel_ref[0] = jnp.int32(0)

    def step(i, carry):
        last, dists = carry
        lx = pxs_ref[last]
        ly = pys_ref[last]
        lz = pzs_ref[last]
        qx_ref[i - 1] = lx
        qy_ref[i - 1] = ly
        qz_ref[i - 1] = lz
        d = (px - lx) ** 2 + (py - ly) ** 2 + (pz - lz) ** 2
        dists = jnp.minimum(dists, d)
        m4 = jnp.maximum(dists[0:4], dists[4:8])
        m2 = jnp.maximum(m4[0:2], m4[2:4])
        mxc = jnp.maximum(m2[0:1], m2[1:2])
        mx = jnp.max(mxc)
        c_star = jnp.min(jnp.where(mxc == mx, lane1, jnp.int32(_FC)))
        rp = jnp.where(dists == mxc, riota, jnp.int32(_FR))
        r4 = jnp.minimum(rp[0:4], rp[4:8])
        r2 = jnp.minimum(r4[0:2], r4[2:4])
        rfirst = jnp.minimum(r2[0:1], r2[1:2])
        r_star = jnp.min(jnp.where(lane1 == c_star, rfirst, jnp.int32(_FR)))
        nxt = c_star * _FR + r_star
        sel_ref[i] = nxt
        return nxt, dists

    last, _ = lax.fori_loop(
        1, M, step, (jnp.int32(0), jnp.full((_FR, _FC), jnp.inf, jnp.float32)))
    qx_ref[M - 1] = pxs_ref[last]
    qy_ref[M - 1] = pys_ref[last]
    qz_ref[M - 1] = pzs_ref[last]


def _fps(px, py, pz, pxs, pys, pzs):
    smem = pl.BlockSpec(memory_space=pltpu.SMEM)
    return pl.pallas_call(
        _fps_body,
        in_specs=[pl.BlockSpec((_FR, _FC), lambda: (0, 0)),
                  pl.BlockSpec((_FR, _FC), lambda: (0, 0)),
                  pl.BlockSpec((_FR, _FC), lambda: (0, 0)),
                  smem, smem, smem],
        out_specs=(smem, smem, smem, smem),
        out_shape=(jax.ShapeDtypeStruct((M,), jnp.int32),
                   jax.ShapeDtypeStruct((M,), jnp.float32),
                   jax.ShapeDtypeStruct((M,), jnp.float32),
                   jax.ShapeDtypeStruct((M,), jnp.float32)),
    )(px, py, pz, pxs, pys, pzs)


def _mlp_body(x_ref, w_ref, b_ref, h_ref):
    h = lax.dot_general(x_ref[...], w_ref[...],
                        (((1,), (1,)), ((), ())),
                        preferred_element_type=jnp.float32)
    h_ref[...] = h + b_ref[...]


def _mlp(x, W, b2d):
    bm = 1024
    return pl.pallas_call(
        _mlp_body,
        grid=(N // bm,),
        in_specs=[pl.BlockSpec((bm, CIN), lambda i: (i, 0)),
                  pl.BlockSpec((COUT, CIN), lambda i: (0, 0)),
                  pl.BlockSpec((1, COUT), lambda i: (0, 0))],
        out_specs=pl.BlockSpec((bm, COUT), lambda i: (i, 0)),
        out_shape=jax.ShapeDtypeStruct((N, COUT), jnp.float32),
    )(x, W, b2d)


_QB = 256


def _knn_body(q_ref, px_ref, py_ref, pz_ref, idx_ref, d2_ref):
    q = q_ref[...]
    qx = q[:, 0:1]
    qy = q[:, 1:2]
    qz = q[:, 2:3]
    d2_ref[...] = ((qx - px_ref[...]) ** 2 + (qy - py_ref[...]) ** 2
                   + (qz - pz_ref[...]) ** 2)
    lanes = lax.broadcasted_iota(jnp.int32, (_QB, N), 1)
    for k in range(K):
        dd = d2_ref[...]
        mn = jnp.min(dd, axis=1, keepdims=True)
        cand = jnp.where(dd == mn, lanes, jnp.int32(N))
        idx = jnp.min(cand, axis=1, keepdims=True)
        idx_ref[:, pl.ds(k, 1)] = idx
        d2_ref[...] = jnp.where(lanes == idx, jnp.inf, dd)


def _knn(qpos, px, py, pz):
    return pl.pallas_call(
        _knn_body,
        grid=(M // _QB,),
        in_specs=[pl.BlockSpec((_QB, 3), lambda i: (i, 0)),
                  pl.BlockSpec((1, N), lambda i: (0, 0)),
                  pl.BlockSpec((1, N), lambda i: (0, 0)),
                  pl.BlockSpec((1, N), lambda i: (0, 0))],
        out_specs=pl.BlockSpec((_QB, K), lambda i: (i, 0)),
        out_shape=jax.ShapeDtypeStruct((M, K), jnp.int32),
        scratch_shapes=[pltpu.VMEM((_QB, N), jnp.float32)],
    )(qpos, px, py, pz)


_NC, _NS, _L = 2, 16, 16
_NW = _NC * _NS
_QPW = M // _NW
_QCH = 4


def _pool_body(h_hbm, idx_hbm, out_hbm, idx_v, rows0, rows1, out_v, sem0, sem1):
    wid = lax.axis_index("s") * _NC + lax.axis_index("c")
    base_q = wid * _QPW
    pltpu.sync_copy(idx_hbm.at[pl.ds(base_q * K, _QPW * K)], idx_v)
    nch = _QPW // _QCH

    def issue(ci, rows, sem):
        return pltpu.async_copy(
            h_hbm.at[idx_v.at[pl.ds(ci * (_QCH * K), _QCH * K)]], rows, sem)

    def drain(ci, rows, sem):
        pltpu.make_async_copy(
            h_hbm.at[idx_v.at[pl.ds(ci * (_QCH * K), _QCH * K)]], rows, sem
        ).wait()

    def compute(ci, rows):
        for ql in range(_QCH):
            for cb in range(COUT // _L):
                acc = rows[ql * K, pl.ds(cb * _L, _L)]
                for r in range(1, K):
                    acc = jnp.maximum(acc, rows[ql * K + r, pl.ds(cb * _L, _L)])
                out_v[ci * _QCH + ql, pl.ds(cb * _L, _L)] = acc

    issue(0, rows0, sem0)

    def pair(p, _):
        ci = p * 2
        drain(ci, rows0, sem0)
        issue(ci + 1, rows1, sem1)
        compute(ci, rows0)
        drain(ci + 1, rows1, sem1)

        @pl.when(ci + 2 < nch)
        def _():
            issue(ci + 2, rows0, sem0)

        compute(ci + 1, rows1)
        return 0

    lax.fori_loop(0, nch // 2, pair, 0)
    pltpu.sync_copy(out_v, out_hbm.at[pl.ds(base_q, _QPW)])


@functools.cache
def _pool_call():
    return pl.kernel(
        _pool_body,
        mesh=plsc.VectorSubcoreMesh(core_axis_name="c", subcore_axis_name="s"),
        out_type=jax.ShapeDtypeStruct((M, COUT), jnp.float32),
        scratch_types=[pltpu.VMEM((_QPW * K,), jnp.int32),
                       pltpu.VMEM((_QCH * K, COUT), jnp.float32),
                       pltpu.VMEM((_QCH * K, COUT), jnp.float32),
                       pltpu.VMEM((_QPW, COUT), jnp.float32),
                       pltpu.SemaphoreType.DMA,
                       pltpu.SemaphoreType.DMA],
    )


def kernel(x, pos, batch, W, b):
    px = pos[:, 0].reshape(_FC, _FR).T
    py = pos[:, 1].reshape(_FC, _FR).T
    pz = pos[:, 2].reshape(_FC, _FR).T
    sel, qx, qy, qz = _fps(px, py, pz, pos[:, 0], pos[:, 1], pos[:, 2])
    sub_pos = jnp.stack([qx, qy, qz], axis=1)

    h = _mlp(x, W, b.reshape(1, COUT))

    idx = _knn(sub_pos,
               pos[:, 0].reshape(1, N),
               pos[:, 1].reshape(1, N),
               pos[:, 2].reshape(1, N))

    out = _pool_call()(h, idx.reshape(-1))

    sub_batch = batch[sel]
    return out, sub_pos, sub_batch


# speedup vs baseline: 1.5583x; 1.5583x over previous
import functools

import jax
import jax.numpy as jnp
import numpy as np
from jax import lax
from jax.experimental import pallas as pl
from jax.experimental.pallas import tpu as pltpu
from jax.experimental.pallas import tpu_sc as plsc

N = 8192
M = 2048
K = 16
CIN = 256
COUT = 512

_FR, _FC = 8, N // 8


def _argmax_pair_tree(dd, ll):
    def fold(da, la, db, lb):
        better = (da > db) | ((da == db) & (la < lb))
        return jnp.where(better, da, db), jnp.where(better, la, lb)

    w = dd.shape[1]
    while w > 1:
        h = w // 2
        dd, ll = fold(dd[:, :h], ll[:, :h], dd[:, h:w], ll[:, h:w])
        w = h
    s = dd.shape[0]
    while s > 1:
        h = s // 2
        dd, ll = fold(dd[:h], ll[:h], dd[h:s], ll[h:s])
        s = h
    return ll[0, 0]


def _fps_body(px_ref, py_ref, pz_ref, pxs_ref, pys_ref, pzs_ref,
              sel_ref, qx_ref, qy_ref, qz_ref):
    px = px_ref[...]
    py = py_ref[...]
    pz = pz_ref[...]
    lin = (lax.broadcasted_iota(jnp.int32, (_FR, _FC), 0) * _FC
           + lax.broadcasted_iota(jnp.int32, (_FR, _FC), 1))

    sel_ref[0] = jnp.int32(0)

    def step(i, carry):
        last, dists = carry
        lx = pxs_ref[last]
        ly = pys_ref[last]
        lz = pzs_ref[last]
        qx_ref[i - 1] = lx
        qy_ref[i - 1] = ly
        qz_ref[i - 1] = lz
        d = (px - lx) ** 2 + (py - ly) ** 2 + (pz - lz) ** 2
        dists = jnp.minimum(dists, d)
        nxt = jnp.argmax(dists.reshape(1, N), axis=1)[0].astype(jnp.int32)
        sel_ref[i] = nxt
        return nxt, dists

    last, _ = lax.fori_loop(
        1, M, step, (jnp.int32(0), jnp.full((_FR, _FC), jnp.inf, jnp.float32)))
    qx_ref[M - 1] = pxs_ref[last]
    qy_ref[M - 1] = pys_ref[last]
    qz_ref[M - 1] = pzs_ref[last]


def _fps(px, py, pz, pxs, pys, pzs):
    smem = pl.BlockSpec(memory_space=pltpu.SMEM)
    return pl.pallas_call(
        _fps_body,
        in_specs=[pl.BlockSpec((_FR, _FC), lambda: (0, 0)),
                  pl.BlockSpec((_FR, _FC), lambda: (0, 0)),
                  pl.BlockSpec((_FR, _FC), lambda: (0, 0)),
                  smem, smem, smem],
        out_specs=(smem, smem, smem, smem),
        out_shape=(jax.ShapeDtypeStruct((M,), jnp.int32),
                   jax.ShapeDtypeStruct((M,), jnp.float32),
                   jax.ShapeDtypeStruct((M,), jnp.float32),
                   jax.ShapeDtypeStruct((M,), jnp.float32)),
    )(px, py, pz, pxs, pys, pzs)


def _mlp_body(x_ref, w_ref, b_ref, h_ref):
    h = lax.dot_general(x_ref[...], w_ref[...],
                        (((1,), (1,)), ((), ())),
                        preferred_element_type=jnp.float32)
    h_ref[...] = h + b_ref[...]


def _mlp(x, W, b2d):
    bm = 1024
    return pl.pallas_call(
        _mlp_body,
        grid=(N // bm,),
        in_specs=[pl.BlockSpec((bm, CIN), lambda i: (i, 0)),
                  pl.BlockSpec((COUT, CIN), lambda i: (0, 0)),
                  pl.BlockSpec((1, COUT), lambda i: (0, 0))],
        out_specs=pl.BlockSpec((bm, COUT), lambda i: (i, 0)),
        out_shape=jax.ShapeDtypeStruct((N, COUT), jnp.float32),
    )(x, W, b2d)


_QB = 256


def _knn_body(q_ref, px_ref, py_ref, pz_ref, idx_ref, d2_ref):
    q = q_ref[...]
    qx = q[:, 0:1]
    qy = q[:, 1:2]
    qz = q[:, 2:3]
    d2_ref[...] = ((qx - px_ref[...]) ** 2 + (qy - py_ref[...]) ** 2
                   + (qz - pz_ref[...]) ** 2)
    lanes = lax.broadcasted_iota(jnp.int32, (_QB, N), 1)
    for k in range(K):
        dd = d2_ref[...]
        mn = jnp.min(dd, axis=1, keepdims=True)
        cand = jnp.where(dd == mn, lanes, jnp.int32(N))
        idx = jnp.min(cand, axis=1, keepdims=True)
        idx_ref[:, pl.ds(k, 1)] = idx
        d2_ref[...] = jnp.where(lanes == idx, jnp.inf, dd)


def _knn(qpos, px, py, pz):
    return pl.pallas_call(
        _knn_body,
        grid=(M // _QB,),
        in_specs=[pl.BlockSpec((_QB, 3), lambda i: (i, 0)),
                  pl.BlockSpec((1, N), lambda i: (0, 0)),
                  pl.BlockSpec((1, N), lambda i: (0, 0)),
                  pl.BlockSpec((1, N), lambda i: (0, 0))],
        out_specs=pl.BlockSpec((_QB, K), lambda i: (i, 0)),
        out_shape=jax.ShapeDtypeStruct((M, K), jnp.int32),
        scratch_shapes=[pltpu.VMEM((_QB, N), jnp.float32)],
    )(qpos, px, py, pz)


_NC, _NS, _L = 2, 16, 16
_NW = _NC * _NS
_QPW = M // _NW
_QCH = 4


def _pool_body(h_hbm, idx_hbm, out_hbm, idx_v, rows0, rows1, out_v, sem0, sem1):
    wid = lax.axis_index("s") * _NC + lax.axis_index("c")
    base_q = wid * _QPW
    pltpu.sync_copy(idx_hbm.at[pl.ds(base_q * K, _QPW * K)], idx_v)
    nch = _QPW // _QCH

    def issue(ci, rows, sem):
        return pltpu.async_copy(
            h_hbm.at[idx_v.at[pl.ds(ci * (_QCH * K), _QCH * K)]], rows, sem)

    def drain(ci, rows, sem):
        pltpu.make_async_copy(
            h_hbm.at[idx_v.at[pl.ds(ci * (_QCH * K), _QCH * K)]], rows, sem
        ).wait()

    def compute(ci, rows):
        for ql in range(_QCH):
            for cb in range(COUT // _L):
                acc = rows[ql * K, pl.ds(cb * _L, _L)]
                for r in range(1, K):
                    acc = jnp.maximum(acc, rows[ql * K + r, pl.ds(cb * _L, _L)])
                out_v[ci * _QCH + ql, pl.ds(cb * _L, _L)] = acc

    issue(0, rows0, sem0)

    def pair(p, _):
        ci = p * 2
        drain(ci, rows0, sem0)
        issue(ci + 1, rows1, sem1)
        compute(ci, rows0)
        drain(ci + 1, rows1, sem1)

        @pl.when(ci + 2 < nch)
        def _():
            issue(ci + 2, rows0, sem0)

        compute(ci + 1, rows1)
        return 0

    lax.fori_loop(0, nch // 2, pair, 0)
    pltpu.sync_copy(out_v, out_hbm.at[pl.ds(base_q, _QPW)])


@functools.cache
def _pool_call():
    return pl.kernel(
        _pool_body,
        mesh=plsc.VectorSubcoreMesh(core_axis_name="c", subcore_axis_name="s"),
        out_type=jax.ShapeDtypeStruct((M, COUT), jnp.float32),
        scratch_types=[pltpu.VMEM((_QPW * K,), jnp.int32),
                       pltpu.VMEM((_QCH * K, COUT), jnp.float32),
                       pltpu.VMEM((_QCH * K, COUT), jnp.float32),
                       pltpu.VMEM((_QPW, COUT), jnp.float32),
                       pltpu.SemaphoreType.DMA,
                       pltpu.SemaphoreType.DMA],
    )


def kernel(x, pos, batch, W, b):
    px = pos[:, 0].reshape(_FR, _FC)
    py = pos[:, 1].reshape(_FR, _FC)
    pz = pos[:, 2].reshape(_FR, _FC)
    sel, qx, qy, qz = _fps(px, py, pz, pos[:, 0], pos[:, 1], pos[:, 2])
    sub_pos = jnp.stack([qx, qy, qz], axis=1)

    h = _mlp(x, W, b.reshape(1, COUT))

    idx = _knn(sub_pos,
               pos[:, 0].reshape(1, N),
               pos[:, 1].reshape(1, N),
               pos[:, 2].reshape(1, N))

    out = _pool_call()(h, idx.reshape(-1))

    sub_batch = batch[sel]
    return out, sub_pos, sub_batch


# speedup vs baseline: 1.5856x; 1.0176x over previous
import functools

import jax
import jax.numpy as jnp
import numpy as np
from jax import lax
from jax.experimental import pallas as pl
from jax.experimental.pallas import tpu as pltpu
from jax.experimental.pallas import tpu_sc as plsc

N = 8192
M = 2048
K = 16
CIN = 256
COUT = 512

_FR, _FC = 8, N // 8


def _argmax_pair_tree(dd, ll):
    def fold(da, la, db, lb):
        better = (da > db) | ((da == db) & (la < lb))
        return jnp.where(better, da, db), jnp.where(better, la, lb)

    w = dd.shape[1]
    while w > 1:
        h = w // 2
        dd, ll = fold(dd[:, :h], ll[:, :h], dd[:, h:w], ll[:, h:w])
        w = h
    s = dd.shape[0]
    while s > 1:
        h = s // 2
        dd, ll = fold(dd[:h], ll[:h], dd[h:s], ll[h:s])
        s = h
    return ll[0, 0]


def _fps_body(px_ref, py_ref, pz_ref, pxs_ref, pys_ref, pzs_ref,
              sel_ref, qx_ref, qy_ref, qz_ref):
    px = px_ref[...]
    py = py_ref[...]
    pz = pz_ref[...]
    lin = (lax.broadcasted_iota(jnp.int32, (_FR, _FC), 0) * _FC
           + lax.broadcasted_iota(jnp.int32, (_FR, _FC), 1))

    sel_ref[0] = jnp.int32(0)

    def step(i, carry):
        last, dists = carry
        lx = pxs_ref[last]
        ly = pys_ref[last]
        lz = pzs_ref[last]
        qx_ref[i - 1] = lx
        qy_ref[i - 1] = ly
        qz_ref[i - 1] = lz
        d = (px - lx) ** 2 + (py - ly) ** 2 + (pz - lz) ** 2
        dists = jnp.minimum(dists, d)
        nxt = jnp.argmax(dists.reshape(1, N), axis=1)[0].astype(jnp.int32)
        sel_ref[i] = nxt
        return nxt, dists

    last, _ = lax.fori_loop(
        1, M, step, (jnp.int32(0), jnp.full((_FR, _FC), jnp.inf, jnp.float32)))
    qx_ref[M - 1] = pxs_ref[last]
    qy_ref[M - 1] = pys_ref[last]
    qz_ref[M - 1] = pzs_ref[last]


def _fps(px, py, pz, pxs, pys, pzs):
    smem = pl.BlockSpec(memory_space=pltpu.SMEM)
    return pl.pallas_call(
        _fps_body,
        in_specs=[pl.BlockSpec((_FR, _FC), lambda: (0, 0)),
                  pl.BlockSpec((_FR, _FC), lambda: (0, 0)),
                  pl.BlockSpec((_FR, _FC), lambda: (0, 0)),
                  smem, smem, smem],
        out_specs=(smem, smem, smem, smem),
        out_shape=(jax.ShapeDtypeStruct((M,), jnp.int32),
                   jax.ShapeDtypeStruct((M,), jnp.float32),
                   jax.ShapeDtypeStruct((M,), jnp.float32),
                   jax.ShapeDtypeStruct((M,), jnp.float32)),
    )(px, py, pz, pxs, pys, pzs)


def _mlp_body(x_ref, w_ref, b_ref, h_ref):
    h = lax.dot_general(x_ref[...], w_ref[...],
                        (((1,), (1,)), ((), ())),
                        preferred_element_type=jnp.float32)
    h_ref[...] = h + b_ref[...]


def _mlp(x, W, b2d):
    bm = 1024
    return pl.pallas_call(
        _mlp_body,
        grid=(N // bm,),
        in_specs=[pl.BlockSpec((bm, CIN), lambda i: (i, 0)),
                  pl.BlockSpec((COUT, CIN), lambda i: (0, 0)),
                  pl.BlockSpec((1, COUT), lambda i: (0, 0))],
        out_specs=pl.BlockSpec((bm, COUT), lambda i: (i, 0)),
        out_shape=jax.ShapeDtypeStruct((N, COUT), jnp.float32),
    )(x, W, b2d)


_QB = 256


def _knn_body(q_ref, px_ref, py_ref, pz_ref, idx_ref, d2_ref):
    q = q_ref[...]
    qx = q[:, 0:1]
    qy = q[:, 1:2]
    qz = q[:, 2:3]
    d2_ref[...] = ((qx - px_ref[...]) ** 2 + (qy - py_ref[...]) ** 2
                   + (qz - pz_ref[...]) ** 2)
    lanes = lax.broadcasted_iota(jnp.int32, (_QB, N), 1)
    for k in range(K):
        dd = d2_ref[...]
        idx = jnp.argmin(dd, axis=1).astype(jnp.int32).reshape(_QB, 1)
        idx_ref[:, pl.ds(k, 1)] = idx
        d2_ref[...] = jnp.where(lanes == idx, jnp.inf, dd)


def _knn(qpos, px, py, pz):
    return pl.pallas_call(
        _knn_body,
        grid=(M // _QB,),
        in_specs=[pl.BlockSpec((_QB, 3), lambda i: (i, 0)),
                  pl.BlockSpec((1, N), lambda i: (0, 0)),
                  pl.BlockSpec((1, N), lambda i: (0, 0)),
                  pl.BlockSpec((1, N), lambda i: (0, 0))],
        out_specs=pl.BlockSpec((_QB, K), lambda i: (i, 0)),
        out_shape=jax.ShapeDtypeStruct((M, K), jnp.int32),
        scratch_shapes=[pltpu.VMEM((_QB, N), jnp.float32)],
    )(qpos, px, py, pz)


_NC, _NS, _L = 2, 16, 16
_NW = _NC * _NS
_QPW = M // _NW
_QCH = 4


def _pool_body(h_hbm, idx_hbm, out_hbm, idx_v, rows0, rows1, out_v, sem0, sem1):
    wid = lax.axis_index("s") * _NC + lax.axis_index("c")
    base_q = wid * _QPW
    pltpu.sync_copy(idx_hbm.at[pl.ds(base_q * K, _QPW * K)], idx_v)
    nch = _QPW // _QCH

    def issue(ci, rows, sem):
        return pltpu.async_copy(
            h_hbm.at[idx_v.at[pl.ds(ci * (_QCH * K), _QCH * K)]], rows, sem)

    def drain(ci, rows, sem):
        pltpu.make_async_copy(
            h_hbm.at[idx_v.at[pl.ds(ci * (_QCH * K), _QCH * K)]], rows, sem
        ).wait()

    def compute(ci, rows):
        for ql in range(_QCH):
            for cb in range(COUT // _L):
                acc = rows[ql * K, pl.ds(cb * _L, _L)]
                for r in range(1, K):
                    acc = jnp.maximum(acc, rows[ql * K + r, pl.ds(cb * _L, _L)])
                out_v[ci * _QCH + ql, pl.ds(cb * _L, _L)] = acc

    issue(0, rows0, sem0)

    def pair(p, _):
        ci = p * 2
        drain(ci, rows0, sem0)
        issue(ci + 1, rows1, sem1)
        compute(ci, rows0)
        drain(ci + 1, rows1, sem1)

        @pl.when(ci + 2 < nch)
        def _():
            issue(ci + 2, rows0, sem0)

        compute(ci + 1, rows1)
        return 0

    lax.fori_loop(0, nch // 2, pair, 0)
    pltpu.sync_copy(out_v, out_hbm.at[pl.ds(base_q, _QPW)])


@functools.cache
def _pool_call():
    return pl.kernel(
        _pool_body,
        mesh=plsc.VectorSubcoreMesh(core_axis_name="c", subcore_axis_name="s"),
        out_type=jax.ShapeDtypeStruct((M, COUT), jnp.float32),
        scratch_types=[pltpu.VMEM((_QPW * K,), jnp.int32),
                       pltpu.VMEM((_QCH * K, COUT), jnp.float32),
                       pltpu.VMEM((_QCH * K, COUT), jnp.float32),
                       pltpu.VMEM((_QPW, COUT), jnp.float32),
                       pltpu.SemaphoreType.DMA,
                       pltpu.SemaphoreType.DMA],
    )


def kernel(x, pos, batch, W, b):
    px = pos[:, 0].reshape(_FR, _FC)
    py = pos[:, 1].reshape(_FR, _FC)
    pz = pos[:, 2].reshape(_FR, _FC)
    sel, qx, qy, qz = _fps(px, py, pz, pos[:, 0], pos[:, 1], pos[:, 2])
    sub_pos = jnp.stack([qx, qy, qz], axis=1)

    h = _mlp(x, W, b.reshape(1, COUT))

    idx = _knn(sub_pos,
               pos[:, 0].reshape(1, N),
               pos[:, 1].reshape(1, N),
               pos[:, 2].reshape(1, N))

    out = _pool_call()(h, idx.reshape(-1))

    sub_batch = batch[sel]
    return out, sub_pos, sub_batch
